# Initial kernel scaffold; baseline (speedup 1.0000x reference)
import jax
import jax.numpy as jnp
from jax.experimental import pallas as pl


def kernel(token_ids, W):
    raise NotImplementedError("write your pallas kernel here")


# speedup vs baseline: 1.7979x; 1.7979x over previous
import functools

import jax
import jax.numpy as jnp
from jax import lax
from jax.experimental import pallas as pl
from jax.experimental.pallas import tpu as pltpu
from jax.experimental.pallas import tpu_sc as plsc

VOCAB = 1000000
D = 64

_info = plsc.get_sparse_core_info()
NC, NS = _info.num_cores, _info.num_subcores
NW = NC * NS

B = 16384 * 50
B_PER_W = B // NW
CHUNK = 512
N_CHUNKS = B_PER_W // CHUNK
SUB = 128
N_SUB = CHUNK // SUB


def _make_kernel():
    mesh = plsc.VectorSubcoreMesh(core_axis_name="c", subcore_axis_name="s")

    @functools.partial(
        pl.kernel,
        mesh=mesh,
        out_type=jax.ShapeDtypeStruct((B, D), jnp.float32),
        scratch_types=[
            pltpu.VMEM((N_SUB, SUB), jnp.int32),
            pltpu.VMEM((CHUNK, D), jnp.float32),
            pltpu.SemaphoreType.DMA,
        ],
        compiler_params=pltpu.CompilerParams(use_tc_tiling_on_sc=False),
    )
    def emb(tok_hbm, w_hbm, out_hbm, idx_v, rows_v, sem):
        wid = lax.axis_index("s") * NC + lax.axis_index("c")
        base = wid * B_PER_W

        def step(i, carry):
            pltpu.sync_copy(tok_hbm.at[wid, i], idx_v)
            for j in range(N_SUB):
                pltpu.async_copy(
                    w_hbm.at[idx_v.at[j]],
                    rows_v.at[pl.ds(j * SUB, SUB)],
                    sem,
                )
            for j in range(N_SUB):
                pltpu.make_async_copy(
                    w_hbm.at[idx_v.at[j]],
                    rows_v.at[pl.ds(j * SUB, SUB)],
                    sem,
                ).wait()
            pltpu.sync_copy(rows_v, out_hbm.at[pl.ds(base + i * CHUNK, CHUNK)])
            return carry

        lax.fori_loop(0, N_CHUNKS, step, 0)

    return emb


_emb = _make_kernel()


@jax.jit
def kernel(token_ids, W):
    toks = token_ids.reshape(NW, N_CHUNKS, N_SUB, SUB)
    out = _emb(toks, W)
    return out.reshape(token_ids.shape + (D,))


# speedup vs baseline: 1.8731x; 1.0418x over previous
import functools

import jax
import jax.numpy as jnp
from jax import lax
from jax.experimental import pallas as pl
from jax.experimental.pallas import tpu as pltpu
from jax.experimental.pallas import tpu_sc as plsc

VOCAB = 1000000
D = 64

_info = plsc.get_sparse_core_info()
NC, NS = _info.num_cores, _info.num_subcores
NW = NC * NS

B = 16384 * 50
B_PER_W = B // NW
CHUNK = 640
N_CHUNKS = B_PER_W // CHUNK
SUB = 128
N_SUB = CHUNK // SUB
NBUF = 2


def _make_kernel():
    mesh = plsc.VectorSubcoreMesh(core_axis_name="c", subcore_axis_name="s")

    @functools.partial(
        pl.kernel,
        mesh=mesh,
        out_type=jax.ShapeDtypeStruct((B, D), jnp.float32),
        scratch_types=[
            [pltpu.VMEM((N_SUB, SUB), jnp.int32) for _ in range(NBUF)],
            [pltpu.VMEM((CHUNK, D), jnp.float32) for _ in range(NBUF)],
            pltpu.SemaphoreType.DMA,
            [pltpu.SemaphoreType.DMA for _ in range(NBUF)],
            [pltpu.SemaphoreType.DMA for _ in range(NBUF)],
        ],
        compiler_params=pltpu.CompilerParams(use_tc_tiling_on_sc=False),
    )
    def emb(tok_hbm, w_hbm, out_hbm, idx_v, rows_v, sem_g, sem_i, sem_o):
        wid = lax.axis_index("s") * NC + lax.axis_index("c")
        base = wid * B_PER_W

        def idx_copy(c, b):
            return pltpu.make_async_copy(tok_hbm.at[wid, c], idx_v[b], sem_i[b])

        def out_copy(c, b):
            return pltpu.make_async_copy(
                rows_v[b], out_hbm.at[pl.ds(base + c * CHUNK, CHUNK)], sem_o[b]
            )

        for b in range(NBUF):
            idx_copy(b, b).start()

        def outer(g, carry):
            for b in range(NBUF):
                i = g * NBUF + b

                @pl.when(g > 0)
                def _wait_rows_free():
                    out_copy(i, b).wait()

                idx_copy(i, b).wait()
                for j in range(N_SUB):
                    pltpu.async_copy(
                        w_hbm.at[idx_v[b].at[j]],
                        rows_v[b].at[pl.ds(j * SUB, SUB)],
                        sem_g,
                    )
                for j in range(N_SUB):
                    pltpu.make_async_copy(
                        w_hbm.at[idx_v[b].at[j]],
                        rows_v[b].at[pl.ds(j * SUB, SUB)],
                        sem_g,
                    ).wait()
                out_copy(i, b).start()

                @pl.when(i + NBUF < N_CHUNKS)
                def _prefetch_idx():
                    idx_copy(i + NBUF, b).start()

            return carry

        lax.fori_loop(0, N_CHUNKS // NBUF, outer, 0)
        for b in range(NBUF):
            out_copy(N_CHUNKS - NBUF + b, b).wait()

    return emb


_emb = _make_kernel()


@jax.jit
def kernel(token_ids, W):
    toks = token_ids.reshape(NW, N_CHUNKS, N_SUB, SUB)
    out = _emb(toks, W)
    return out.reshape(token_ids.shape + (D,))


# speedup vs baseline: 1.8744x; 1.0007x over previous
import functools

import jax
import jax.numpy as jnp
from jax import lax
from jax.experimental import pallas as pl
from jax.experimental.pallas import tpu as pltpu
from jax.experimental.pallas import tpu_sc as plsc

VOCAB = 1000000
D = 64

_info = plsc.get_sparse_core_info()
NC, NS = _info.num_cores, _info.num_subcores
NW = NC * NS

B = 16384 * 50
B_PER_W = B // NW
CHUNK = 640
N_CHUNKS = B_PER_W // CHUNK
SUB = 128
N_SUB = CHUNK // SUB
NBUF = 2


def _make_kernel():
    mesh = plsc.VectorSubcoreMesh(core_axis_name="c", subcore_axis_name="s")

    @functools.partial(
        pl.kernel,
        mesh=mesh,
        out_type=jax.ShapeDtypeStruct((B, D), jnp.float32),
        scratch_types=[
            [pltpu.VMEM((CHUNK,), jnp.int32) for _ in range(NBUF)],
            [pltpu.VMEM((CHUNK, D), jnp.float32) for _ in range(NBUF)],
            pltpu.SemaphoreType.DMA,
            [pltpu.SemaphoreType.DMA for _ in range(NBUF)],
            [pltpu.SemaphoreType.DMA for _ in range(NBUF)],
        ],
        compiler_params=pltpu.CompilerParams(use_tc_tiling_on_sc=False),
    )
    def emb(tok_hbm, w_hbm, out_hbm, idx_v, rows_v, sem_g, sem_i, sem_o):
        wid = lax.axis_index("s") * NC + lax.axis_index("c")
        base = wid * B_PER_W

        def idx_copy(c, b):
            return pltpu.make_async_copy(tok_hbm.at[wid, c], idx_v[b], sem_i[b])

        def out_copy(c, b):
            return pltpu.make_async_copy(
                rows_v[b], out_hbm.at[pl.ds(base + c * CHUNK, CHUNK)], sem_o[b]
            )

        for b in range(NBUF):
            idx_copy(b, b).start()

        def outer(g, carry):
            for b in range(NBUF):
                i = g * NBUF + b

                @pl.when(g > 0)
                def _wait_rows_free():
                    out_copy(i, b).wait()

                idx_copy(i, b).wait()
                pltpu.async_copy(w_hbm.at[idx_v[b]], rows_v[b], sem_g)
                pltpu.make_async_copy(w_hbm.at[idx_v[b]], rows_v[b], sem_g).wait()
                out_copy(i, b).start()

                @pl.when(i + NBUF < N_CHUNKS)
                def _prefetch_idx():
                    idx_copy(i + NBUF, b).start()

            return carry

        lax.fori_loop(0, N_CHUNKS // NBUF, outer, 0)
        for b in range(NBUF):
            out_copy(N_CHUNKS - NBUF + b, b).wait()

    return emb


_emb = _make_kernel()


@jax.jit
def kernel(token_ids, W):
    toks = token_ids.reshape(NW, N_CHUNKS, CHUNK)
    out = _emb(toks, W)
    return out.reshape(token_ids.shape + (D,))
